# Initial kernel scaffold; baseline (speedup 1.0000x reference)
import jax
import jax.numpy as jnp
from jax.experimental import pallas as pl


def kernel(p0, x0, p1, x1, W0, b0, W1, b1, Wc, bc):
    raise NotImplementedError("write your pallas kernel here")


# speedup vs baseline: 1.1501x; 1.1501x over previous
import jax
import jax.numpy as jnp
from jax import lax
from jax.experimental import pallas as pl
from jax.experimental.pallas import tpu as pltpu
from jax.experimental.pallas import tpu_sc as plsc

_N0, _N1, _D0, _D1, _DF, _KC = 16384, 4096, 128, 256, 128, 13
_B0 = 512
_NB0 = _N0 // _B0
_B1 = 512
_NB1 = _N1 // _B1

_NW = 32
_RPW = _N0 // _NW
_CHUNK = 128
_NCH = _RPW // _CHUNK

_BF = jnp.bfloat16
_F32 = jnp.float32


def _mlp_nn_body(p0_ref, p1t_ref, x0_ref, w0_ref, b0_ref, f0_ref, idx_ref):
    f0 = jnp.dot(x0_ref[...].astype(_BF), w0_ref[...].astype(_BF),
                 preferred_element_type=_F32)
    f0_ref[...] = jnp.maximum(f0 + b0_ref[...], 0.0).astype(_BF).astype(_F32)
    p0b = p0_ref[...]
    p1b = p1t_ref[...]
    px, py, pz = p0b[:, 0:1], p0b[:, 1:2], p0b[:, 2:3]
    qx, qy, qz = p1b[0:1, :], p1b[1:2, :], p1b[2:3, :]
    p0sq = (px * px + pz * pz) + py * py
    p1sq = (qx * qx + qz * qz) + qy * qy
    dot2 = jnp.dot(p0b * 2.0, p1b, preferred_element_type=_F32)
    d = (p0sq - dot2) + p1sq
    minv = jnp.min(d, axis=1, keepdims=True)
    iota = lax.broadcasted_iota(jnp.int32, d.shape, 1)
    cand = jnp.where(d == minv, iota, jnp.int32(_N1))
    idx_ref[...] = jnp.min(cand, axis=1, keepdims=True)


def _mlp1_body(x1_ref, w1_ref, b1_ref, f1_ref):
    f1 = jnp.dot(x1_ref[...].astype(_BF), w1_ref[...].astype(_BF),
                 preferred_element_type=_F32)
    f1_ref[...] = jnp.maximum(f1 + b1_ref[...], 0.0).astype(_BF).astype(_F32)


def _head_body(f0_ref, up1_ref, wc_ref, bc_ref, out_ref):
    xcat = jnp.concatenate([f0_ref[...], up1_ref[...]], axis=1).astype(_BF)
    out = jnp.dot(xcat, wc_ref[...].astype(_BF), preferred_element_type=_F32)
    out_ref[...] = out + bc_ref[...]


def _sc_gather_body(f1_hbm, idx_hbm, up1_hbm, idx_v, rows_v, sem):
    wid = lax.axis_index("s") * 2 + lax.axis_index("c")
    pltpu.sync_copy(idx_hbm.at[pl.ds(wid * _NCH, _NCH)], idx_v)
    descs = [
        pltpu.async_copy(
            f1_hbm.at[idx_v.at[t]],
            rows_v.at[pl.ds(t * _CHUNK, _CHUNK)],
            sem,
        )
        for t in range(_NCH)
    ]
    for d_ in descs:
        d_.wait()
    pltpu.sync_copy(rows_v, up1_hbm.at[pl.ds(wid * _RPW, _RPW)])


def kernel(p0, x0, p1, x1, W0, b0, W1, b1, Wc, bc):
    p0p = jnp.pad(p0, ((0, 0), (0, 5)))
    p1t = jnp.pad(p1, ((0, 0), (0, 5))).T
    b0r = b0.reshape(1, _DF)
    b1r = b1.reshape(1, _DF)
    bcr = bc.reshape(1, _KC)

    f0, idx = pl.pallas_call(
        _mlp_nn_body,
        grid=(_NB0,),
        in_specs=[
            pl.BlockSpec((_B0, 8), lambda i: (i, 0)),
            pl.BlockSpec((8, _N1), lambda i: (0, 0)),
            pl.BlockSpec((_B0, _D0), lambda i: (i, 0)),
            pl.BlockSpec((_D0, _DF), lambda i: (0, 0)),
            pl.BlockSpec((1, _DF), lambda i: (0, 0)),
        ],
        out_specs=[
            pl.BlockSpec((_B0, _DF), lambda i: (i, 0)),
            pl.BlockSpec((_B0, 1), lambda i: (i, 0)),
        ],
        out_shape=[
            jax.ShapeDtypeStruct((_N0, _DF), jnp.float32),
            jax.ShapeDtypeStruct((_N0, 1), jnp.int32),
        ],
    )(p0p, p1t, x0, W0, b0r)

    f1 = pl.pallas_call(
        _mlp1_body,
        grid=(_NB1,),
        in_specs=[
            pl.BlockSpec((_B1, _D1), lambda i: (i, 0)),
            pl.BlockSpec((_D1, _DF), lambda i: (0, 0)),
            pl.BlockSpec((1, _DF), lambda i: (0, 0)),
        ],
        out_specs=pl.BlockSpec((_B1, _DF), lambda i: (i, 0)),
        out_shape=jax.ShapeDtypeStruct((_N1, _DF), jnp.float32),
    )(x1, W1, b1r)

    idx2d = idx.reshape(_NW * _NCH, _CHUNK)

    mesh = plsc.VectorSubcoreMesh(core_axis_name="c", subcore_axis_name="s")
    up1 = pl.kernel(
        _sc_gather_body,
        out_type=jax.ShapeDtypeStruct((_N0, _DF), jnp.float32),
        mesh=mesh,
        scratch_types=[
            pltpu.VMEM((_NCH, _CHUNK), jnp.int32),
            pltpu.VMEM((_RPW, _DF), jnp.float32),
            pltpu.SemaphoreType.DMA,
        ],
    )(f1, idx2d)

    out = pl.pallas_call(
        _head_body,
        grid=(_NB0,),
        in_specs=[
            pl.BlockSpec((_B0, _DF), lambda i: (i, 0)),
            pl.BlockSpec((_B0, _DF), lambda i: (i, 0)),
            pl.BlockSpec((2 * _DF, _KC), lambda i: (0, 0)),
            pl.BlockSpec((1, _KC), lambda i: (0, 0)),
        ],
        out_specs=pl.BlockSpec((_B0, _KC), lambda i: (i, 0)),
        out_shape=jax.ShapeDtypeStruct((_N0, _KC), jnp.float32),
    )(f0, up1, Wc, bcr)
    return out
